# Initial kernel scaffold; baseline (speedup 1.0000x reference)
import jax
import jax.numpy as jnp
from jax.experimental import pallas as pl


def kernel(h, pos, edge_index, W1, b1, W2, b2, W3, b3, W4, b4):
    raise NotImplementedError("write your pallas kernel here")


# speedup vs baseline: 2.3923x; 2.3923x over previous
import dataclasses
import functools
import math

import jax
import jax.numpy as jnp
from jax import lax
from jax.experimental import pallas as pl
from jax.experimental.pallas import tpu as pltpu
from jax.experimental.pallas import tpu_sc as plsc

HIDDEN = 128
NUM_RBF = 50
CUTOFF = 5.0
N_NODES = 10000
N_EDGES = 320000
WIDTH = CUTOFF / (NUM_RBF - 1)

NC, NS = 2, 16
NW = NC * NS
EPT = N_EDGES // NW
ROWS_PT = N_NODES // NS

_CHUNK_A = 2000
_CB = 80

def _silu(x):
    return x * jax.nn.sigmoid(x)


@functools.lru_cache(maxsize=None)
def _sc_kernels():
    mesh = plsc.VectorSubcoreMesh(core_axis_name="c", subcore_axis_name="s",
                                  num_cores=NC, num_subcores=NS)
    sc_params = dataclasses.replace(pltpu.CompilerParams(),
                                    needs_layout_passes=False)

    @functools.partial(
        pl.kernel,
        out_type=jax.ShapeDtypeStruct((N_EDGES,), jnp.float32),
        mesh=mesh,
        compiler_params=sc_params,
        scratch_types=[
            pltpu.VMEM((N_NODES * 3,), jnp.float32),
            pltpu.VMEM((_CHUNK_A,), jnp.int32),
            pltpu.VMEM((_CHUNK_A,), jnp.int32),
            pltpu.VMEM((_CHUNK_A,), jnp.float32),
        ],
    )
    def dist2(pos_hbm, row_hbm, col_hbm, d2_hbm, pos_v, row_v, col_v, d2_v):
        wid = lax.axis_index("s") * NC + lax.axis_index("c")
        base = wid * EPT
        pltpu.sync_copy(pos_hbm, pos_v)

        @pl.loop(0, EPT, step=_CHUNK_A)
        def _(off):
            pltpu.sync_copy(row_hbm.at[pl.ds(base + off, _CHUNK_A)], row_v)
            pltpu.sync_copy(col_hbm.at[pl.ds(base + off, _CHUNK_A)], col_v)

            @pl.loop(0, _CHUNK_A, step=16)
            def _(i):
                rv = row_v[pl.ds(i, 16)] * 3
                cv = col_v[pl.ds(i, 16)] * 3
                dx = plsc.load_gather(pos_v, [rv]) - plsc.load_gather(pos_v, [cv])
                dy = plsc.load_gather(pos_v, [rv + 1]) - plsc.load_gather(pos_v, [cv + 1])
                dz = plsc.load_gather(pos_v, [rv + 2]) - plsc.load_gather(pos_v, [cv + 2])
                d2_v[pl.ds(i, 16)] = dx * dx + dy * dy + dz * dz

            pltpu.sync_copy(d2_v, d2_hbm.at[pl.ds(base + off, _CHUNK_A)])

    @functools.partial(
        pl.kernel,
        out_type=jax.ShapeDtypeStruct((NC, NS, ROWS_PT, HIDDEN), jnp.float32),
        mesh=mesh,
        compiler_params=sc_params,
        scratch_types=[
            pltpu.VMEM((_CB,), jnp.int32),
            pltpu.VMEM((_CB,), jnp.int32),
            pltpu.VMEM((_CB, HIDDEN), jnp.float32),
            pltpu.VMEM((_CB, HIDDEN), jnp.float32),
            pltpu.VMEM_SHARED((N_NODES, HIDDEN), jnp.float32),
        ],
    )
    def msg_agg(h_hbm, wf_hbm, row_hbm, col_hbm, out_hbm,
                row_v, col_v, hrows, wrows, agg_sh):
        c = lax.axis_index("c")
        s = lax.axis_index("s")
        wid = s * NC + c
        base = wid * EPT

        @pl.loop(0, _CB)
        def _(i):
            for j in range(HIDDEN // 16):
                wrows[i, pl.ds(j * 16, 16)] = jnp.zeros((16,), jnp.float32)

        _off = 0
        while _off < ROWS_PT:
            _n = min(_CB, ROWS_PT - _off)
            pltpu.sync_copy(wrows.at[pl.ds(0, _n)],
                            agg_sh.at[pl.ds(s * ROWS_PT + _off, _n)])
            _off += _n
        plsc.subcore_barrier()

        @pl.loop(0, EPT, step=_CB)
        def _(off):
            pltpu.sync_copy(row_hbm.at[pl.ds(base + off, _CB)], row_v)
            pltpu.sync_copy(col_hbm.at[pl.ds(base + off, _CB)], col_v)
            pltpu.sync_copy(h_hbm.at[col_v], hrows)
            pltpu.sync_copy(wf_hbm.at[pl.ds(base + off, _CB)], wrows)

            @pl.loop(0, _CB)
            def _(i):
                for j in range(HIDDEN // 16):
                    sl = pl.ds(j * 16, 16)
                    hrows[i, sl] = hrows[i, sl] * wrows[i, sl]

            pltpu.sync_copy(hrows, agg_sh.at[row_v], add=True)

        plsc.subcore_barrier()
        pltpu.sync_copy(agg_sh.at[pl.ds(s * ROWS_PT, ROWS_PT)],
                        out_hbm.at[c, s])

    return dist2, msg_agg


_BE = 2000


def _filter_body(d2_ref, W1_ref, b1_ref, W2_ref, b2_ref, out_ref):
    d = jnp.sqrt(d2_ref[...] + 1e-8)
    z = d / WIDTH - lax.broadcasted_iota(jnp.int32, (1, NUM_RBF), 1).astype(jnp.float32)
    rbf = jnp.exp(-0.5 * z * z)
    hmid = _silu(jnp.dot(rbf, W1_ref[...], preferred_element_type=jnp.float32)
                 + b1_ref[...])
    filt = jnp.dot(hmid, W2_ref[...], preferred_element_type=jnp.float32) + b2_ref[...]
    cut = 0.5 * (jnp.cos(math.pi / CUTOFF * d) + 1.0)
    cut = cut * (d <= CUTOFF).astype(jnp.float32)
    out_ref[...] = filt * cut


def _filter(d2, W1, b1, W2, b2):
    return pl.pallas_call(
        _filter_body,
        grid=(N_EDGES // _BE,),
        in_specs=[
            pl.BlockSpec((_BE, 1), lambda i: (i, 0)),
            pl.BlockSpec((NUM_RBF, HIDDEN), lambda i: (0, 0)),
            pl.BlockSpec((1, HIDDEN), lambda i: (0, 0)),
            pl.BlockSpec((HIDDEN, HIDDEN), lambda i: (0, 0)),
            pl.BlockSpec((1, HIDDEN), lambda i: (0, 0)),
        ],
        out_specs=pl.BlockSpec((_BE, HIDDEN), lambda i: (i, 0)),
        out_shape=jax.ShapeDtypeStruct((N_EDGES, HIDDEN), jnp.float32),
    )(d2, W1, b1, W2, b2)


_BN = 1000


def _node_body(h_ref, p0_ref, p1_ref, W3_ref, b3_ref, W4_ref, b4_ref, out_ref):
    agg = p0_ref[...] + p1_ref[...]
    t = _silu(jnp.dot(agg, W3_ref[...], preferred_element_type=jnp.float32)
              + b3_ref[...])
    out_ref[...] = (h_ref[...]
                    + jnp.dot(t, W4_ref[...], preferred_element_type=jnp.float32)
                    + b4_ref[...])


def _node(h, p0, p1, W3, b3, W4, b4):
    return pl.pallas_call(
        _node_body,
        grid=(N_NODES // _BN,),
        in_specs=[
            pl.BlockSpec((_BN, HIDDEN), lambda i: (i, 0)),
            pl.BlockSpec((_BN, HIDDEN), lambda i: (i, 0)),
            pl.BlockSpec((_BN, HIDDEN), lambda i: (i, 0)),
            pl.BlockSpec((HIDDEN, HIDDEN), lambda i: (0, 0)),
            pl.BlockSpec((1, HIDDEN), lambda i: (0, 0)),
            pl.BlockSpec((HIDDEN, HIDDEN), lambda i: (0, 0)),
            pl.BlockSpec((1, HIDDEN), lambda i: (0, 0)),
        ],
        out_specs=pl.BlockSpec((_BN, HIDDEN), lambda i: (i, 0)),
        out_shape=jax.ShapeDtypeStruct((N_NODES, HIDDEN), jnp.float32),
    )(h, p0, p1, W3, b3, W4, b4)


def kernel(h, pos, edge_index, W1, b1, W2, b2, W3, b3, W4, b4):
    row = edge_index[0].astype(jnp.int32)
    col = edge_index[1].astype(jnp.int32)
    pos_flat = jnp.reshape(pos.astype(jnp.float32), (-1,))
    dist2, msg_agg = _sc_kernels()
    d2 = dist2(pos_flat, row, col)
    wf = _filter(d2.reshape(N_EDGES, 1), W1, b1.reshape(1, HIDDEN),
                 W2, b2.reshape(1, HIDDEN))
    parts = msg_agg(h, wf, row, col).reshape(NC, N_NODES, HIDDEN)
    return _node(h, parts[0], parts[1], W3, b3.reshape(1, HIDDEN),
                 W4, b4.reshape(1, HIDDEN))


# speedup vs baseline: 6.1496x; 2.5706x over previous
import dataclasses
import functools
import math

import jax
import jax.numpy as jnp
from jax import lax
from jax.experimental import pallas as pl
from jax.experimental.pallas import tpu as pltpu
from jax.experimental.pallas import tpu_sc as plsc

HIDDEN = 128
NUM_RBF = 50
CUTOFF = 5.0
N_NODES = 10000
N_EDGES = 320000
WIDTH = CUTOFF / (NUM_RBF - 1)

NC, NS = 2, 16
NW = NC * NS
EPT = N_EDGES // NW
ROWS_PT = N_NODES // NS

_CHUNK_A = 2000
_CB = 40
NCH = EPT // _CB
_QB = 2

def _silu(x):
    return x * jax.nn.sigmoid(x)


@functools.lru_cache(maxsize=None)
def _sc_kernels():
    mesh = plsc.VectorSubcoreMesh(core_axis_name="c", subcore_axis_name="s",
                                  num_cores=NC, num_subcores=NS)
    sc_params = dataclasses.replace(pltpu.CompilerParams(),
                                    needs_layout_passes=False)

    @functools.partial(
        pl.kernel,
        out_type=jax.ShapeDtypeStruct((N_EDGES,), jnp.float32),
        mesh=mesh,
        compiler_params=sc_params,
        scratch_types=[
            pltpu.VMEM((N_NODES * 3,), jnp.float32),
            pltpu.VMEM((_CHUNK_A,), jnp.int32),
            pltpu.VMEM((_CHUNK_A,), jnp.int32),
            pltpu.VMEM((_CHUNK_A,), jnp.float32),
        ],
    )
    def dist2(pos_hbm, row_hbm, col_hbm, d2_hbm, pos_v, row_v, col_v, d2_v):
        wid = lax.axis_index("s") * NC + lax.axis_index("c")
        base = wid * EPT
        pltpu.sync_copy(pos_hbm, pos_v)

        @pl.loop(0, EPT, step=_CHUNK_A)
        def _(off):
            pltpu.sync_copy(row_hbm.at[pl.ds(base + off, _CHUNK_A)], row_v)
            pltpu.sync_copy(col_hbm.at[pl.ds(base + off, _CHUNK_A)], col_v)

            @pl.loop(0, _CHUNK_A, step=16)
            def _(i):
                rv = row_v[pl.ds(i, 16)] * 3
                cv = col_v[pl.ds(i, 16)] * 3
                dx = plsc.load_gather(pos_v, [rv]) - plsc.load_gather(pos_v, [cv])
                dy = plsc.load_gather(pos_v, [rv + 1]) - plsc.load_gather(pos_v, [cv + 1])
                dz = plsc.load_gather(pos_v, [rv + 2]) - plsc.load_gather(pos_v, [cv + 2])
                d2_v[pl.ds(i, 16)] = dx * dx + dy * dy + dz * dz

            pltpu.sync_copy(d2_v, d2_hbm.at[pl.ds(base + off, _CHUNK_A)])

    @functools.partial(
        pl.kernel,
        out_type=jax.ShapeDtypeStruct((NC, NS, ROWS_PT, HIDDEN), jnp.float32),
        mesh=mesh,
        compiler_params=sc_params,
        scratch_types=[
            [pltpu.VMEM((2 * _CB,), jnp.int32) for _ in range(2)],
            [pltpu.VMEM((_CB,), jnp.int32) for _ in range(_QB)],
            [pltpu.VMEM((_CB, HIDDEN), jnp.float32) for _ in range(_QB)],
            [pltpu.VMEM((_CB, HIDDEN), jnp.float32) for _ in range(_QB)],
            pltpu.VMEM((2 * _CB, HIDDEN), jnp.float32),
            pltpu.VMEM_SHARED((N_NODES, HIDDEN), jnp.float32),
            pltpu.SemaphoreType.DMA,
            pltpu.SemaphoreType.DMA,
            pltpu.SemaphoreType.DMA,
            pltpu.SemaphoreType.DMA,
            pltpu.SemaphoreType.DMA,
        ],
    )
    def msg_agg(h_hbm, wf_hbm, row_hbm, col_hbm, out_hbm,
                row_v, col_v, hbufs, wbufs, mbuf, agg_sh,
                gsem, wsem, ssem, isem, rsem):
        c = lax.axis_index("c")
        s = lax.axis_index("s")
        wid = s * NC + c
        base = wid * EPT
        NPAIR = NCH // 2

        @pl.loop(0, 2 * _CB)
        def _(i):
            for j in range(HIDDEN // 16):
                mbuf[i, pl.ds(j * 16, 16)] = jnp.zeros((16,), jnp.float32)

        _off = 0
        while _off < ROWS_PT:
            _n = min(2 * _CB, ROWS_PT - _off)
            pltpu.sync_copy(mbuf.at[pl.ds(0, _n)],
                            agg_sh.at[pl.ds(s * ROWS_PT + _off, _n)])
            _off += _n
        plsc.subcore_barrier()

        for k in range(_QB):
            pltpu.async_copy(col_hbm.at[pl.ds(base + k * _CB, _CB)],
                             col_v[k], isem)

        @pl.loop(0, NPAIR)
        def _(t):
            j0 = t * 2
            even = (t & 1) == 0
            odd = jnp.logical_not(even)

            for k in range(_QB):
                pltpu.make_async_copy(col_hbm.at[pl.ds(base, _CB)],
                                      col_v[k], isem).wait()
            gcps = [pltpu.async_copy(h_hbm.at[col_v[k]], hbufs[k], gsem)
                    for k in range(_QB)]

            @pl.when(even)
            def _():
                pltpu.async_copy(
                    row_hbm.at[pl.ds(base + j0 * _CB, 2 * _CB)], row_v[0],
                    rsem)

            @pl.when(odd)
            def _():
                pltpu.async_copy(
                    row_hbm.at[pl.ds(base + j0 * _CB, 2 * _CB)], row_v[1],
                    rsem)

            wcps = [pltpu.async_copy(
                wf_hbm.at[pl.ds(base + (j0 + k) * _CB, _CB)], wbufs[k], wsem)
                for k in range(_QB)]

            @pl.when(t > 0)
            def _():
                pltpu.make_async_copy(mbuf, agg_sh.at[row_v[0]], ssem).wait()

            for k in range(_QB):
                gcps[k].wait()
                wcps[k].wait()

            @pl.when(t + 1 < NPAIR)
            def _():
                for k in range(_QB):
                    pltpu.async_copy(
                        col_hbm.at[pl.ds(base + (j0 + 2 + k) * _CB, _CB)],
                        col_v[k], isem)

            for k in range(_QB):
                hb, wb = hbufs[k], wbufs[k]

                @pl.loop(0, _CB)
                def _(i):
                    for jc in range(HIDDEN // 16):
                        sl = pl.ds(jc * 16, 16)
                        mbuf[k * _CB + i, sl] = hb[i, sl] * wb[i, sl]

            pltpu.make_async_copy(row_hbm.at[pl.ds(base, 2 * _CB)], row_v[0],
                                  rsem).wait()
            last = t == NPAIR - 1

            @pl.when(jnp.logical_and(even, jnp.logical_not(last)))
            def _():
                pltpu.async_copy(mbuf, agg_sh.at[row_v[0]], ssem, add=True)

            @pl.when(jnp.logical_and(odd, jnp.logical_not(last)))
            def _():
                pltpu.async_copy(mbuf, agg_sh.at[row_v[1]], ssem, add=True)

            @pl.when(last)
            def _():
                pltpu.sync_copy(mbuf, agg_sh.at[row_v[0]], add=True)

        plsc.subcore_barrier()
        pltpu.sync_copy(agg_sh.at[pl.ds(s * ROWS_PT, ROWS_PT)],
                        out_hbm.at[c, s])

    return dist2, msg_agg


_BE = 2000

_CUT_COEF = [
    1.0,
    -4.934802200544679,
    4.058712126416768,
    -1.3352627688545895,
    0.2353306303588932,
    -0.02580689139001406,
    0.0019295743094039231,
    -0.00010463810492484565,
    4.303069587032081e-06,
    -1.3878952462213771e-07,
    3.5943139226189845e-09,
]


def _filter_body(d2_ref, W1_ref, b1_ref, W2_ref, b2_ref, out_ref):
    d2r = d2_ref[...].reshape(1, _BE)
    u = d2r * (1.0 / (CUTOFF * CUTOFF))
    acc = jnp.full_like(u, _CUT_COEF[-1])
    for cc in _CUT_COEF[-2::-1]:
        acc = acc * u + cc
    cutr = 0.5 * (acc + 1.0)
    cutr = jnp.where(d2r <= CUTOFF * CUTOFF, cutr, 0.0)
    dwr = jnp.sqrt(d2r + 1e-8) * (0.7071067811865476 / WIDTH)
    cols = jnp.transpose(jnp.concatenate([dwr, cutr], axis=0))
    dw = cols[:, 0:1]
    cut = cols[:, 1:2]
    ks = (lax.broadcasted_iota(jnp.int32, (1, NUM_RBF), 1).astype(jnp.float32)
          * 0.7071067811865476)
    z = dw - ks
    rbf = jnp.exp(z * (ks - dw))
    hmid = _silu(jnp.dot(rbf, W1_ref[...], preferred_element_type=jnp.float32)
                 + b1_ref[...])
    filt = jnp.dot(hmid, W2_ref[...], preferred_element_type=jnp.float32) + b2_ref[...]
    out_ref[...] = filt * cut


def _filter(d2, W1, b1, W2, b2):
    return pl.pallas_call(
        _filter_body,
        grid=(N_EDGES // _BE,),
        in_specs=[
            pl.BlockSpec((1, 1, _BE), lambda i: (i, 0, 0)),
            pl.BlockSpec((NUM_RBF, HIDDEN), lambda i: (0, 0)),
            pl.BlockSpec((1, HIDDEN), lambda i: (0, 0)),
            pl.BlockSpec((HIDDEN, HIDDEN), lambda i: (0, 0)),
            pl.BlockSpec((1, HIDDEN), lambda i: (0, 0)),
        ],
        out_specs=pl.BlockSpec((_BE, HIDDEN), lambda i: (i, 0)),
        out_shape=jax.ShapeDtypeStruct((N_EDGES, HIDDEN), jnp.float32),
    )(d2, W1, b1, W2, b2)


_BN = 1000


def _node_body(h_ref, p0_ref, p1_ref, W3_ref, b3_ref, W4_ref, b4_ref, out_ref):
    agg = p0_ref[...] + p1_ref[...]
    t = _silu(jnp.dot(agg, W3_ref[...], preferred_element_type=jnp.float32)
              + b3_ref[...])
    out_ref[...] = (h_ref[...]
                    + jnp.dot(t, W4_ref[...], preferred_element_type=jnp.float32)
                    + b4_ref[...])


def _node(h, p0, p1, W3, b3, W4, b4):
    return pl.pallas_call(
        _node_body,
        grid=(N_NODES // _BN,),
        in_specs=[
            pl.BlockSpec((_BN, HIDDEN), lambda i: (i, 0)),
            pl.BlockSpec((_BN, HIDDEN), lambda i: (i, 0)),
            pl.BlockSpec((_BN, HIDDEN), lambda i: (i, 0)),
            pl.BlockSpec((HIDDEN, HIDDEN), lambda i: (0, 0)),
            pl.BlockSpec((1, HIDDEN), lambda i: (0, 0)),
            pl.BlockSpec((HIDDEN, HIDDEN), lambda i: (0, 0)),
            pl.BlockSpec((1, HIDDEN), lambda i: (0, 0)),
        ],
        out_specs=pl.BlockSpec((_BN, HIDDEN), lambda i: (i, 0)),
        out_shape=jax.ShapeDtypeStruct((N_NODES, HIDDEN), jnp.float32),
    )(h, p0, p1, W3, b3, W4, b4)


def kernel(h, pos, edge_index, W1, b1, W2, b2, W3, b3, W4, b4):
    row = edge_index[0].astype(jnp.int32)
    col = edge_index[1].astype(jnp.int32)
    pos_flat = jnp.reshape(pos.astype(jnp.float32), (-1,))
    dist2, msg_agg = _sc_kernels()
    d2 = dist2(pos_flat, row, col)
    wf = _filter(d2.reshape(N_EDGES // _BE, 1, _BE), W1, b1.reshape(1, HIDDEN),
                 W2, b2.reshape(1, HIDDEN))
    parts = msg_agg(h, wf, row, col).reshape(NC, N_NODES, HIDDEN)
    return _node(h, parts[0], parts[1], W3, b3.reshape(1, HIDDEN),
                 W4, b4.reshape(1, HIDDEN))
